# Initial kernel scaffold; baseline (speedup 1.0000x reference)
import jax
import jax.numpy as jnp
from jax.experimental import pallas as pl


def kernel(input_ids, word_table, pos_table):
    raise NotImplementedError("write your pallas kernel here")


# speedup vs baseline: 5.9882x; 5.9882x over previous
import functools

import jax
import jax.numpy as jnp
from jax import lax
from jax.experimental import pallas as pl
from jax.experimental.pallas import tpu as pltpu
from jax.experimental.pallas import tpu_sc as plsc


def _build_sc_kernel(total, L, E, n_workers, num_cores):
    per_w = total // n_workers
    chunk = L
    n_chunk = per_w // chunk
    lanes = E // 16

    mesh = plsc.VectorSubcoreMesh(core_axis_name="c", subcore_axis_name="s")

    @functools.partial(
        pl.kernel,
        out_type=jax.ShapeDtypeStruct((total, E), jnp.float32),
        mesh=mesh,
        scratch_types=[
            pltpu.VMEM((chunk,), jnp.int32),
            pltpu.VMEM((chunk, E), jnp.float32),
            pltpu.VMEM((L, E), jnp.float32),
            pltpu.SemaphoreType.DMA,
        ],
        compiler_params=pltpu.CompilerParams(use_tc_tiling_on_sc=False),
    )
    def emb(ids_hbm, wt_hbm, pos_hbm, out_hbm, ids_v, rows_v, pos_v, sem):
        cid = lax.axis_index("c")
        sid = lax.axis_index("s")
        wid = sid * num_cores + cid
        base = wid * per_w

        pltpu.sync_copy(pos_hbm.at[pl.ds(0, L)], pos_v)

        def chunk_body(i, carry):
            off = base + i * chunk
            pltpu.sync_copy(ids_hbm.at[pl.ds(off, chunk)], ids_v)
            pltpu.async_copy(wt_hbm.at[ids_v], rows_v, sem).wait()

            def row_body(r, carry2):
                for j in range(lanes):
                    sl = pl.ds(j * 16, 16)
                    rows_v[r, sl] = rows_v[r, sl] + pos_v[r, sl]
                return carry2

            lax.fori_loop(0, chunk, row_body, 0)
            pltpu.sync_copy(rows_v, out_hbm.at[pl.ds(off, chunk)])
            return carry

        lax.fori_loop(0, n_chunk, chunk_body, 0)

    return emb


def kernel(input_ids, word_table, pos_table):
    B, L = input_ids.shape
    E = word_table.shape[1]
    info = plsc.get_sparse_core_info()
    n_workers = info.num_cores * info.num_subcores
    total = B * L

    ids_flat = input_ids.reshape(total).astype(jnp.int32)
    emb = _build_sc_kernel(total, L, E, n_workers, info.num_cores)
    out = emb(ids_flat, word_table, pos_table)
    return out.reshape(B, L, E)


# speedup vs baseline: 7.6689x; 1.2807x over previous
import functools

import jax
import jax.numpy as jnp
from jax import lax
from jax.experimental import pallas as pl
from jax.experimental.pallas import tpu as pltpu
from jax.experimental.pallas import tpu_sc as plsc


def _build_sc_kernel(total, L, E, n_workers, num_cores):
    per_w = total // n_workers
    chunk = L
    n_chunk = per_w // chunk
    n_pair = n_chunk // 2
    lanes = E // 16
    row_unroll = 4

    mesh = plsc.VectorSubcoreMesh(core_axis_name="c", subcore_axis_name="s")

    @functools.partial(
        pl.kernel,
        out_type=jax.ShapeDtypeStruct((total, E), jnp.float32),
        mesh=mesh,
        scratch_types=[
            pltpu.VMEM((per_w,), jnp.int32),
            pltpu.VMEM((chunk, E), jnp.float32),
            pltpu.VMEM((chunk, E), jnp.float32),
            pltpu.VMEM((L, E), jnp.float32),
            pltpu.SemaphoreType.DMA,
            pltpu.SemaphoreType.DMA,
            pltpu.SemaphoreType.DMA,
            pltpu.SemaphoreType.DMA,
        ],
        compiler_params=pltpu.CompilerParams(use_tc_tiling_on_sc=False),
    )
    def emb(ids_hbm, wt_hbm, pos_hbm, out_hbm, ids_v, rows0, rows1, pos_v,
            g0, g1, s0, s1):
        cid = lax.axis_index("c")
        sid = lax.axis_index("s")
        wid = sid * num_cores + cid
        base = wid * per_w

        pltpu.sync_copy(pos_hbm.at[pl.ds(0, L)], pos_v)
        pltpu.sync_copy(ids_hbm.at[pl.ds(base, per_w)], ids_v)

        def gather(i, rows, gsem):
            return pltpu.make_async_copy(
                wt_hbm.at[ids_v.at[pl.ds(i * chunk, chunk)]], rows, gsem)

        def store(i, rows, ssem):
            return pltpu.make_async_copy(
                rows, out_hbm.at[pl.ds(base + i * chunk, chunk)], ssem)

        def add_pos(rows):
            def grp_body(g, carry):
                r0 = g * row_unroll
                for rr in range(row_unroll):
                    r = r0 + rr
                    for j in range(lanes):
                        sl = pl.ds(j * 16, 16)
                        rows[r, sl] = rows[r, sl] + pos_v[r, sl]
                return carry
            lax.fori_loop(0, chunk // row_unroll, grp_body, 0)

        def step(i, rows_a, rows_b, gsem_a, gsem_b, ssem_a, ssem_b):
            gather(i, rows_a, gsem_a).wait()

            @pl.when(i > 0)
            def _():
                store(i - 1, rows_b, ssem_b).wait()

            @pl.when(i + 1 < n_chunk)
            def _():
                gather(i + 1, rows_b, gsem_b).start()

            add_pos(rows_a)
            store(i, rows_a, ssem_a).start()

        gather(0, rows0, g0).start()

        def pair_body(k, carry):
            step(2 * k, rows0, rows1, g0, g1, s0, s1)
            step(2 * k + 1, rows1, rows0, g1, g0, s1, s0)
            return carry

        lax.fori_loop(0, n_pair, pair_body, 0)
        store(n_chunk - 1, rows1, s1).wait()

    return emb


def kernel(input_ids, word_table, pos_table):
    B, L = input_ids.shape
    E = word_table.shape[1]
    info = plsc.get_sparse_core_info()
    n_workers = info.num_cores * info.num_subcores
    total = B * L

    ids_flat = input_ids.reshape(total).astype(jnp.int32)
    emb = _build_sc_kernel(total, L, E, n_workers, info.num_cores)
    out = emb(ids_flat, word_table, pos_table)
    return out.reshape(B, L, E)


# speedup vs baseline: 7.6698x; 1.0001x over previous
import functools

import jax
import jax.numpy as jnp
from jax import lax
from jax.experimental import pallas as pl
from jax.experimental.pallas import tpu as pltpu
from jax.experimental.pallas import tpu_sc as plsc


def _build_sc_kernel(B, L, E, n_workers, num_cores):
    per_w = B // n_workers
    n_pair = per_w // 2
    lanes = E // 16
    row_unroll = 4

    mesh = plsc.VectorSubcoreMesh(core_axis_name="c", subcore_axis_name="s")

    @functools.partial(
        pl.kernel,
        out_type=jax.ShapeDtypeStruct((B, L, E), jnp.float32),
        mesh=mesh,
        scratch_types=[
            pltpu.VMEM((per_w, L), jnp.int32),
            pltpu.VMEM((L, E), jnp.float32),
            pltpu.VMEM((L, E), jnp.float32),
            pltpu.VMEM((L, E), jnp.float32),
            pltpu.SemaphoreType.DMA,
            pltpu.SemaphoreType.DMA,
            pltpu.SemaphoreType.DMA,
            pltpu.SemaphoreType.DMA,
        ],
        compiler_params=pltpu.CompilerParams(use_tc_tiling_on_sc=False),
    )
    def emb(ids_hbm, wt_hbm, pos_hbm, out_hbm, ids_v, rows0, rows1, pos_v,
            g0, g1, s0, s1):
        cid = lax.axis_index("c")
        sid = lax.axis_index("s")
        wid = sid * num_cores + cid
        base = wid * per_w

        pltpu.sync_copy(pos_hbm.at[pl.ds(0, L)], pos_v)
        pltpu.sync_copy(ids_hbm.at[pl.ds(base, per_w)], ids_v)

        def gather(i, rows, gsem):
            return pltpu.make_async_copy(wt_hbm.at[ids_v.at[i]], rows, gsem)

        def store(i, rows, ssem):
            return pltpu.make_async_copy(rows, out_hbm.at[base + i], ssem)

        def add_pos(rows):
            def grp_body(g, carry):
                r0 = g * row_unroll
                for rr in range(row_unroll):
                    r = r0 + rr
                    for j in range(lanes):
                        sl = pl.ds(j * 16, 16)
                        rows[r, sl] = rows[r, sl] + pos_v[r, sl]
                return carry
            lax.fori_loop(0, L // row_unroll, grp_body, 0)

        def step(i, rows_a, rows_b, gsem_a, gsem_b, ssem_a, ssem_b):
            gather(i, rows_a, gsem_a).wait()

            @pl.when(i > 0)
            def _():
                store(i - 1, rows_b, ssem_b).wait()

            @pl.when(i + 1 < per_w)
            def _():
                gather(i + 1, rows_b, gsem_b).start()

            add_pos(rows_a)
            store(i, rows_a, ssem_a).start()

        gather(0, rows0, g0).start()

        def pair_body(k, carry):
            step(2 * k, rows0, rows1, g0, g1, s0, s1)
            step(2 * k + 1, rows1, rows0, g1, g0, s1, s0)
            return carry

        lax.fori_loop(0, n_pair, pair_body, 0)
        store(per_w - 1, rows1, s1).wait()

    return emb


def kernel(input_ids, word_table, pos_table):
    B, L = input_ids.shape
    E = word_table.shape[1]
    info = plsc.get_sparse_core_info()
    n_workers = info.num_cores * info.num_subcores

    emb = _build_sc_kernel(B, L, E, n_workers, info.num_cores)
    return emb(input_ids.astype(jnp.int32), word_table, pos_table)
